# baseline (speedup 1.0000x reference)
import functools

import jax
import jax.numpy as jnp
from jax import lax
from jax.experimental import pallas as pl
from jax.experimental.pallas import tpu as pltpu
from jax.experimental.pallas import tpu_sc as plsc

NC = 2
NS = 16
NW = NC * NS
L = 16

VOCAB = 1000000
DIM = 64
MET_ROW = 5

B_SEQ = 4096
L_SEQ = 200
N = B_SEQ * L_SEQ
BB = B_SEQ // NW
SLAB = BB * L_SEQ

PLANE = DIM * B_SEQ
GSTRIDE = 8 * B_SEQ
TILE = 8 * 128
NGROUPS = DIM // 8


def _body(table_hbm, seq_hbm, met_hbm, out_hbm,
          seq_v, s_v, idx2, s2, rows_v, tile_v, row5_v, gsems, osems):
    wid = lax.axis_index("s") * NC + lax.axis_index("c")
    base0 = wid * SLAB

    pltpu.sync_copy(table_hbm.at[pl.ds(MET_ROW, 1), :], row5_v)
    r5 = [row5_v[0, pl.ds(q * L, L)] for q in range(DIM // L)]

    pltpu.sync_copy(seq_hbm.at[pl.ds(base0, SLAB)], seq_v)
    pltpu.sync_copy(met_hbm.at[pl.ds(base0, SLAB)], s_v)

    iota_l = lax.iota(jnp.int32, L) * L_SEQ
    scat = [lax.iota(jnp.int32, L) * 128 + q * (L * 128)
            for q in range(DIM // L)]

    def build(l, b):
        for g in range(BB // L):
            iv = iota_l + (g * L * L_SEQ + l)
            sv = plsc.load_gather(seq_v, [iv])
            mv = plsc.load_gather(s_v, [iv])
            keep = sv != 0
            idx2[b, pl.ds(g * L, L)] = jnp.where(keep, sv, MET_ROW)
            s2[pl.ds(b * BB + g * L, L)] = jnp.where(keep, mv, -1.0)

    def fire_gather(b):
        pltpu.async_copy(
            table_hbm.at[idx2.at[b]], rows_v.at[b], gsems.at[b])

    def wait_gather(b):
        pltpu.make_async_copy(
            table_hbm.at[idx2.at[b]], rows_v.at[b], gsems.at[b]).wait()

    def compute(b):
        off = b * (DIM * 128)

        def rowfn(i, _):
            sb = plsc.load_gather(
                s2, [jnp.full((L,), b * BB + i, jnp.int32)])
            for q in range(DIM // L):
                v = rows_v[b, i, pl.ds(q * L, L)]
                plsc.store_scatter(
                    tile_v, [scat[q] + (off + i)], v + sb * r5[q])
            return 0

        lax.fori_loop(0, BB, rowfn, 0, unroll=4)

    def out_copies(l, b):
        off = b * (DIM * 128)
        return [
            pltpu.make_async_copy(
                tile_v.at[pl.ds(off + g * TILE, TILE)],
                out_hbm.at[pl.ds(l * PLANE + g * GSTRIDE + wid * TILE,
                                 TILE)],
                osems.at[b],
            )
            for g in range(NGROUPS)
        ]

    build(0, 0)
    fire_gather(0)

    def step(l, _):
        b = lax.rem(l, 2)
        nb = 1 - b

        @pl.when(l + 1 < L_SEQ)
        def _prefetch():
            @pl.when(l >= 1)
            def _drain_prev_out():
                for cp in out_copies(l - 1, nb):
                    cp.wait()
            build(l + 1, nb)
            fire_gather(nb)

        wait_gather(b)
        compute(b)
        for cp in out_copies(l, b):
            cp.start()
        return 0

    lax.fori_loop(0, L_SEQ, step, 0)

    for cp in out_copies(L_SEQ - 2, lax.rem(L_SEQ - 2, 2)):
        cp.wait()
    for cp in out_copies(L_SEQ - 1, lax.rem(L_SEQ - 1, 2)):
        cp.wait()


@jax.jit
def _run(table, seq_f, met_f):
    mesh = plsc.VectorSubcoreMesh(
        core_axis_name="c", subcore_axis_name="s",
        num_cores=NC, num_subcores=NS,
    )
    f = pl.kernel(
        _body,
        out_type=jax.ShapeDtypeStruct((L_SEQ * PLANE,), jnp.float32),
        mesh=mesh,
        compiler_params=pltpu.CompilerParams(
            needs_layout_passes=False, use_tc_tiling_on_sc=False,
        ),
        scratch_types=[
            pltpu.VMEM((SLAB,), jnp.int32),
            pltpu.VMEM((SLAB,), jnp.float32),
            pltpu.VMEM((2, BB), jnp.int32),
            pltpu.VMEM((2 * BB,), jnp.float32),
            pltpu.VMEM((2, BB, DIM), jnp.float32),
            pltpu.VMEM((2 * DIM * 128,), jnp.float32),
            pltpu.VMEM((1, DIM), jnp.float32),
            pltpu.SemaphoreType.DMA((2,)),
            pltpu.SemaphoreType.DMA((2,)),
        ],
    )
    return f(table, seq_f, met_f)


def kernel(seq, met, table):
    seq_f = seq.reshape(N)
    met_f = met.reshape(N)
    flat = _run(table, seq_f, met_f)
    a = flat.reshape(L_SEQ, NGROUPS, NW, 8, 128)
    a = a.transpose(2, 4, 0, 1, 3)
    return a.reshape(B_SEQ, L_SEQ, DIM)


# speedup vs baseline: 1.4672x; 1.4672x over previous
import functools

import jax
import jax.numpy as jnp
from jax import lax
from jax.experimental import pallas as pl
from jax.experimental.pallas import tpu as pltpu
from jax.experimental.pallas import tpu_sc as plsc

NC = 2
NS = 16
NW = NC * NS
L = 16

VOCAB = 1000000
DIM = 64
MET_ROW = 5

B_SEQ = 4096
L_SEQ = 200
N = B_SEQ * L_SEQ
BB = B_SEQ // NW
SLAB = BB * L_SEQ

PLANE = DIM * B_SEQ
GSTRIDE = 8 * B_SEQ
TILE = 8 * 128
NGROUPS = DIM // 8


def _body(table_hbm, seq_hbm, met_hbm, out_hbm,
          seq_v, s_v, idx2, s2, rows_v, tile_v, row5_v, gsems, osems):
    wid = lax.axis_index("s") * NC + lax.axis_index("c")
    base0 = wid * SLAB

    pltpu.sync_copy(table_hbm.at[pl.ds(MET_ROW, 1), :], row5_v)
    r5 = [row5_v[0, pl.ds(q * L, L)] for q in range(DIM // L)]

    pltpu.sync_copy(seq_hbm.at[pl.ds(base0, SLAB)], seq_v)
    pltpu.sync_copy(met_hbm.at[pl.ds(base0, SLAB)], s_v)

    iota_l = lax.iota(jnp.int32, L) * L_SEQ
    dim_i = lax.iota(jnp.int32, L)
    scf = [(dim_i + q * L) * 128 for q in range(DIM // L)]

    def build(l, b):
        for g in range(BB // L):
            iv = iota_l + (g * L * L_SEQ + l)
            sv = plsc.load_gather(seq_v, [iv])
            mv = plsc.load_gather(s_v, [iv])
            keep = sv != 0
            idx2[b, pl.ds(g * L, L)] = jnp.where(keep, sv, MET_ROW)
            s2[pl.ds(b * BB + g * L, L)] = jnp.where(keep, mv, -1.0)

    def fire_gather(b):
        pltpu.async_copy(
            table_hbm.at[idx2.at[b]], rows_v.at[b], gsems.at[b])

    def wait_gather(b):
        pltpu.make_async_copy(
            table_hbm.at[idx2.at[b]], rows_v.at[b], gsems.at[b]).wait()

    def compute(b):
        bfull = jnp.full((L,), b, jnp.int32)

        def diagfn(i0, _):
            t = (i0 + dim_i) & (BB - 1)
            sd = plsc.load_gather(s2, [t + b * BB])
            boff = t + b * (DIM * 128)
            for q in range(DIM // L):
                v = plsc.load_gather(rows_v, [bfull, t, dim_i + q * L])
                plsc.store_scatter(
                    tile_v, [scf[q] + boff], v + sd * r5[q])
            return 0

        lax.fori_loop(0, BB, diagfn, 0, unroll=4)

    def out_copies(l, b):
        off = b * (DIM * 128)
        return [
            pltpu.make_async_copy(
                tile_v.at[pl.ds(off + g * TILE, TILE)],
                out_hbm.at[pl.ds(l * PLANE + g * GSTRIDE + wid * TILE,
                                 TILE)],
                osems.at[b],
            )
            for g in range(NGROUPS)
        ]

    def drain_out(b):
        pltpu.make_async_copy(
            out_hbm.at[pl.ds(0, DIM * 128)],
            tile_v.at[pl.ds(b * (DIM * 128), DIM * 128)],
            osems.at[b],
        ).wait()

    build(0, 0)
    fire_gather(0)

    def step(l, _):
        b = lax.rem(l, 2)
        nb = 1 - b

        @pl.when(l + 1 < L_SEQ)
        def _prefetch():
            @pl.when(l >= 1)
            def _drain_prev_out():
                drain_out(nb)
            build(l + 1, nb)
            fire_gather(nb)

        wait_gather(b)
        compute(b)
        for cp in out_copies(l, b):
            cp.start()
        return 0

    lax.fori_loop(0, L_SEQ, step, 0)

    drain_out(0)
    drain_out(1)


@jax.jit
def _run(table, seq_f, met_f):
    mesh = plsc.VectorSubcoreMesh(
        core_axis_name="c", subcore_axis_name="s",
        num_cores=NC, num_subcores=NS,
    )
    f = pl.kernel(
        _body,
        out_type=jax.ShapeDtypeStruct((L_SEQ * PLANE,), jnp.float32),
        mesh=mesh,
        compiler_params=pltpu.CompilerParams(
            needs_layout_passes=False, use_tc_tiling_on_sc=False,
        ),
        scratch_types=[
            pltpu.VMEM((SLAB,), jnp.int32),
            pltpu.VMEM((SLAB,), jnp.float32),
            pltpu.VMEM((2, BB), jnp.int32),
            pltpu.VMEM((2 * BB,), jnp.float32),
            pltpu.VMEM((2, BB, DIM), jnp.float32),
            pltpu.VMEM((2 * DIM * 128,), jnp.float32),
            pltpu.VMEM((1, DIM), jnp.float32),
            pltpu.SemaphoreType.DMA((2,)),
            pltpu.SemaphoreType.DMA((2,)),
        ],
    )
    return f(table, seq_f, met_f)


def kernel(seq, met, table):
    seq_f = seq.reshape(N)
    met_f = met.reshape(N)
    flat = _run(table, seq_f, met_f)
    a = flat.reshape(L_SEQ, NGROUPS, NW, 8, 128)
    a = a.transpose(2, 4, 0, 1, 3)
    return a.reshape(B_SEQ, L_SEQ, DIM)


# speedup vs baseline: 1.4711x; 1.0026x over previous
import functools

import jax
import jax.numpy as jnp
from jax import lax
from jax.experimental import pallas as pl
from jax.experimental.pallas import tpu as pltpu
from jax.experimental.pallas import tpu_sc as plsc

NC = 2
NS = 16
NW = NC * NS
L = 16

VOCAB = 1000000
DIM = 64
MET_ROW = 5

B_SEQ = 4096
L_SEQ = 200
N = B_SEQ * L_SEQ
BB = B_SEQ // NW
SLAB = BB * L_SEQ

PLANE = DIM * B_SEQ
GSTRIDE = 8 * B_SEQ
TILE = 8 * 128
NGROUPS = DIM // 8


def _body(table_hbm, seq_hbm, met_hbm, out_hbm,
          seq_v, s_v, idx2, s2, rows_v, tile_v, row5_v, gsems, osems):
    wid = lax.axis_index("s") * NC + lax.axis_index("c")
    base0 = wid * SLAB

    pltpu.sync_copy(table_hbm.at[pl.ds(MET_ROW, 1), :], row5_v)
    r5 = [row5_v[0, pl.ds(q * L, L)] for q in range(DIM // L)]

    pltpu.sync_copy(seq_hbm.at[pl.ds(base0, SLAB)], seq_v)
    pltpu.sync_copy(met_hbm.at[pl.ds(base0, SLAB)], s_v)

    iota_l = lax.iota(jnp.int32, L) * L_SEQ
    dim_i = lax.iota(jnp.int32, L)
    scf = [(dim_i + q * L) * 128 for q in range(DIM // L)]

    def build(l, b):
        for g in range(BB // L):
            iv = iota_l + (g * L * L_SEQ + l)
            sv = plsc.load_gather(seq_v, [iv])
            mv = plsc.load_gather(s_v, [iv])
            keep = sv != 0
            idx2[b, pl.ds(g * L, L)] = jnp.where(keep, sv, MET_ROW)
            s2[pl.ds(b * BB + g * L, L)] = jnp.where(keep, mv, -1.0)

    def fire_gather(b):
        pltpu.async_copy(
            table_hbm.at[idx2.at[b]], rows_v.at[b], gsems.at[b])

    def wait_gather(b):
        pltpu.make_async_copy(
            table_hbm.at[idx2.at[b]], rows_v.at[b], gsems.at[b]).wait()

    def compute(b):
        bfull = jnp.full((L,), b, jnp.int32)

        def diagfn(i0, _):
            t = (i0 + dim_i) & (BB - 1)
            sd = plsc.load_gather(s2, [t + b * BB])
            boff = t + b * (DIM * 128)
            for q in range(DIM // L):
                v = plsc.load_gather(rows_v, [bfull, t, dim_i + q * L])
                plsc.store_scatter(
                    tile_v, [scf[q] + boff], v + sd * r5[q])
            return 0

        lax.fori_loop(0, BB, diagfn, 0, unroll=8)

    def out_copies(l, b):
        off = b * (DIM * 128)
        return [
            pltpu.make_async_copy(
                tile_v.at[pl.ds(off + g * TILE, TILE)],
                out_hbm.at[pl.ds(l * PLANE + g * GSTRIDE + wid * TILE,
                                 TILE)],
                osems.at[b],
            )
            for g in range(NGROUPS)
        ]

    def drain_out(b):
        pltpu.make_async_copy(
            out_hbm.at[pl.ds(0, DIM * 128)],
            tile_v.at[pl.ds(b * (DIM * 128), DIM * 128)],
            osems.at[b],
        ).wait()

    build(0, 0)
    fire_gather(0)

    def step(l, _):
        b = lax.rem(l, 2)
        nb = 1 - b

        @pl.when(l + 1 < L_SEQ)
        def _prefetch():
            @pl.when(l >= 1)
            def _drain_prev_out():
                drain_out(nb)
            build(l + 1, nb)
            fire_gather(nb)

        wait_gather(b)
        compute(b)
        for cp in out_copies(l, b):
            cp.start()
        return 0

    lax.fori_loop(0, L_SEQ, step, 0)

    drain_out(0)
    drain_out(1)


@jax.jit
def _run(table, seq_f, met_f):
    mesh = plsc.VectorSubcoreMesh(
        core_axis_name="c", subcore_axis_name="s",
        num_cores=NC, num_subcores=NS,
    )
    f = pl.kernel(
        _body,
        out_type=jax.ShapeDtypeStruct((L_SEQ * PLANE,), jnp.float32),
        mesh=mesh,
        compiler_params=pltpu.CompilerParams(
            needs_layout_passes=False, use_tc_tiling_on_sc=False,
        ),
        scratch_types=[
            pltpu.VMEM((SLAB,), jnp.int32),
            pltpu.VMEM((SLAB,), jnp.float32),
            pltpu.VMEM((2, BB), jnp.int32),
            pltpu.VMEM((2 * BB,), jnp.float32),
            pltpu.VMEM((2, BB, DIM), jnp.float32),
            pltpu.VMEM((2 * DIM * 128,), jnp.float32),
            pltpu.VMEM((1, DIM), jnp.float32),
            pltpu.SemaphoreType.DMA((2,)),
            pltpu.SemaphoreType.DMA((2,)),
        ],
    )
    return f(table, seq_f, met_f)


def kernel(seq, met, table):
    seq_f = seq.reshape(N)
    met_f = met.reshape(N)
    flat = _run(table, seq_f, met_f)
    a = flat.reshape(L_SEQ, NGROUPS, NW, 8, 128)
    a = a.transpose(2, 4, 0, 1, 3)
    return a.reshape(B_SEQ, L_SEQ, DIM)


# speedup vs baseline: 1.4713x; 1.0002x over previous
import jax
import jax.numpy as jnp
from jax import lax
from jax.experimental import pallas as pl
from jax.experimental.pallas import tpu as pltpu
from jax.experimental.pallas import tpu_sc as plsc

NC = 2
NS = 16
NW = NC * NS
L = 16

VOCAB = 1000000
DIM = 64
MET_ROW = 5

B_SEQ = 4096
L_SEQ = 200
N = B_SEQ * L_SEQ
BB = B_SEQ // NW
SLAB = BB * L_SEQ

PLANE = DIM * B_SEQ
GSTRIDE = 8 * B_SEQ
TILE = 8 * 128
NGROUPS = DIM // 8


def _body(table_hbm, seq_hbm, met_hbm, out_hbm,
          seq_v, s_v, idx2, s2, rows_v, tile_v, row5_v, gsems, osems):
    wid = lax.axis_index("s") * NC + lax.axis_index("c")
    base0 = wid * SLAB

    pltpu.sync_copy(table_hbm.at[pl.ds(MET_ROW, 1), :], row5_v)
    r5 = [row5_v[0, pl.ds(q * L, L)] for q in range(DIM // L)]

    pltpu.sync_copy(seq_hbm.at[pl.ds(base0, SLAB)], seq_v)
    pltpu.sync_copy(met_hbm.at[pl.ds(base0, SLAB)], s_v)

    iota_l = lax.iota(jnp.int32, L) * L_SEQ
    dim_i = lax.iota(jnp.int32, L)
    scf = [(dim_i + q * L) * 128 for q in range(DIM // L)]

    def build(l, b):
        for g in range(BB // L):
            iv = iota_l + (g * L * L_SEQ + l)
            sv = plsc.load_gather(seq_v, [iv])
            mv = plsc.load_gather(s_v, [iv])
            keep = sv != 0
            idx2[b, pl.ds(g * L, L)] = jnp.where(keep, sv, MET_ROW)
            s2[pl.ds(b * BB + g * L, L)] = jnp.where(keep, mv, -1.0)

    def fire_gather(b):
        pltpu.async_copy(
            table_hbm.at[idx2.at[b]], rows_v.at[b], gsems.at[b])

    def wait_gather(b):
        pltpu.make_async_copy(
            table_hbm.at[idx2.at[b]], rows_v.at[b], gsems.at[b]).wait()

    def compute(b):
        bfull = jnp.full((L,), b, jnp.int32)

        def diagfn(i0, _):
            t = (i0 + dim_i) & (BB - 1)
            sd = plsc.load_gather(s2, [t + b * BB])
            boff = t + b * (DIM * 128)
            for q in range(DIM // L):
                v = plsc.load_gather(rows_v, [bfull, t, dim_i + q * L])
                plsc.store_scatter(
                    tile_v, [scf[q] + boff], v + sd * r5[q])
            return 0

        lax.fori_loop(0, BB, diagfn, 0, unroll=8)

    def out_copies(l, b):
        off = b * (DIM * 128)
        return [
            pltpu.make_async_copy(
                tile_v.at[pl.ds(off + g * TILE, TILE)],
                out_hbm.at[pl.ds(l * PLANE + g * GSTRIDE + wid * TILE,
                                 TILE)],
                osems.at[b],
            )
            for g in range(NGROUPS)
        ]

    def drain_out(b):
        pltpu.make_async_copy(
            out_hbm.at[pl.ds(0, DIM * 128)],
            tile_v.at[pl.ds(b * (DIM * 128), DIM * 128)],
            osems.at[b],
        ).wait()

    build(0, 0)
    fire_gather(0)

    def step(l, _):
        b = lax.rem(l, 2)
        nb = 1 - b

        @pl.when(l + 1 < L_SEQ)
        def _prefetch():
            @pl.when(l >= 1)
            def _drain_prev_out():
                drain_out(nb)
            build(l + 1, nb)
            fire_gather(nb)

        wait_gather(b)
        compute(b)
        for cp in out_copies(l, b):
            cp.start()
        return 0

    lax.fori_loop(0, L_SEQ, step, 0)

    drain_out(0)
    drain_out(1)


@jax.jit
def _run(table, seq_f, met_f):
    mesh = plsc.VectorSubcoreMesh(
        core_axis_name="c", subcore_axis_name="s",
        num_cores=NC, num_subcores=NS,
    )
    f = pl.kernel(
        _body,
        out_type=jax.ShapeDtypeStruct((L_SEQ * PLANE,), jnp.float32),
        mesh=mesh,
        compiler_params=pltpu.CompilerParams(
            needs_layout_passes=False, use_tc_tiling_on_sc=False,
        ),
        scratch_types=[
            pltpu.VMEM((SLAB,), jnp.int32),
            pltpu.VMEM((SLAB,), jnp.float32),
            pltpu.VMEM((2, BB), jnp.int32),
            pltpu.VMEM((2 * BB,), jnp.float32),
            pltpu.VMEM((2, BB, DIM), jnp.float32),
            pltpu.VMEM((2 * DIM * 128,), jnp.float32),
            pltpu.VMEM((1, DIM), jnp.float32),
            pltpu.SemaphoreType.DMA((2,)),
            pltpu.SemaphoreType.DMA((2,)),
        ],
    )
    return f(table, seq_f, met_f)


def kernel(seq, met, table):
    seq_f = seq.reshape(N)
    met_f = met.reshape(N)
    flat = _run(table, seq_f, met_f)
    a = flat.reshape(L_SEQ, NGROUPS, NW, 8, 128)
    a = a.transpose(2, 4, 0, 1, 3)
    return a.reshape(B_SEQ, L_SEQ, DIM)


# speedup vs baseline: 2.3239x; 1.5795x over previous
import functools

import jax
import jax.numpy as jnp
from jax import lax
from jax.experimental import pallas as pl
from jax.experimental.pallas import tpu as pltpu
from jax.experimental.pallas import tpu_sc as plsc

NC = 2
NS = 16
NW = NC * NS
L = 16

VOCAB = 1000000
DIM = 64
MET_ROW = 5

B_SEQ = 4096
L_SEQ = 200
N = B_SEQ * L_SEQ
BB = B_SEQ // NW
SLAB = BB * L_SEQ

PLANE = DIM * B_SEQ
GSTRIDE = 8 * B_SEQ
TILE = 8 * 128
NGROUPS = DIM // 8


def _body(table_hbm, seq_hbm, met_hbm, out_hbm,
          seq_v, s_v, idx2, s2, rows_v, tile_v, row5_v, gsems, osems):
    wid = lax.axis_index("s") * NC + lax.axis_index("c")
    base0 = wid * SLAB

    pltpu.sync_copy(table_hbm.at[pl.ds(MET_ROW, 1), :], row5_v)
    r5 = [row5_v[0, pl.ds(q * L, L)] for q in range(DIM // L)]

    pltpu.sync_copy(seq_hbm.at[pl.ds(base0, SLAB)], seq_v)
    pltpu.sync_copy(met_hbm.at[pl.ds(base0, SLAB)], s_v)

    iota_l = lax.iota(jnp.int32, L) * L_SEQ
    dim_i = lax.iota(jnp.int32, L)
    scf = [(dim_i + q * L) * 128 for q in range(DIM // L)]

    def build(l, b):
        for g in range(BB // L):
            iv = iota_l + (g * L * L_SEQ + l)
            sv = plsc.load_gather(seq_v, [iv])
            mv = plsc.load_gather(s_v, [iv])
            keep = sv != 0
            idx2[b, pl.ds(g * L, L)] = jnp.where(keep, sv, MET_ROW)
            s2[pl.ds(b * BB + g * L, L)] = jnp.where(keep, mv, -1.0)

    def fire_gather(b):
        pltpu.async_copy(
            table_hbm.at[idx2.at[b]], rows_v.at[b], gsems.at[b])

    def wait_gather(b):
        pltpu.make_async_copy(
            table_hbm.at[idx2.at[b]], rows_v.at[b], gsems.at[b]).wait()

    def compute(b):
        bfull = jnp.full((L,), b, jnp.int32)

        @functools.partial(plsc.parallel_loop, 0, BB, unroll=8)
        def diagfn(i0):
            t = (i0 + dim_i) & (BB - 1)
            sd = plsc.load_gather(s2, [t + b * BB])
            boff = t + b * (DIM * 128)
            for q in range(DIM // L):
                v = plsc.load_gather(rows_v, [bfull, t, dim_i + q * L])
                plsc.store_scatter(
                    tile_v, [scf[q] + boff], v + sd * r5[q])

    def out_copies(l, b):
        off = b * (DIM * 128)
        return [
            pltpu.make_async_copy(
                tile_v.at[pl.ds(off + g * TILE, TILE)],
                out_hbm.at[pl.ds(l * PLANE + g * GSTRIDE + wid * TILE,
                                 TILE)],
                osems.at[b],
            )
            for g in range(NGROUPS)
        ]

    def drain_out(b):
        pltpu.make_async_copy(
            out_hbm.at[pl.ds(0, DIM * 128)],
            tile_v.at[pl.ds(b * (DIM * 128), DIM * 128)],
            osems.at[b],
        ).wait()

    build(0, 0)
    fire_gather(0)

    def step(l, _):
        b = lax.rem(l, 2)
        nb = 1 - b

        @pl.when(l + 1 < L_SEQ)
        def _prefetch():
            @pl.when(l >= 1)
            def _drain_prev_out():
                drain_out(nb)
            build(l + 1, nb)
            fire_gather(nb)

        wait_gather(b)
        compute(b)
        for cp in out_copies(l, b):
            cp.start()
        return 0

    lax.fori_loop(0, L_SEQ, step, 0)

    drain_out(0)
    drain_out(1)


@jax.jit
def _run(table, seq_f, met_f):
    mesh = plsc.VectorSubcoreMesh(
        core_axis_name="c", subcore_axis_name="s",
        num_cores=NC, num_subcores=NS,
    )
    f = pl.kernel(
        _body,
        out_type=jax.ShapeDtypeStruct((L_SEQ * PLANE,), jnp.float32),
        mesh=mesh,
        compiler_params=pltpu.CompilerParams(
            needs_layout_passes=False, use_tc_tiling_on_sc=False,
        ),
        scratch_types=[
            pltpu.VMEM((SLAB,), jnp.int32),
            pltpu.VMEM((SLAB,), jnp.float32),
            pltpu.VMEM((2, BB), jnp.int32),
            pltpu.VMEM((2 * BB,), jnp.float32),
            pltpu.VMEM((2, BB, DIM), jnp.float32),
            pltpu.VMEM((2 * DIM * 128,), jnp.float32),
            pltpu.VMEM((1, DIM), jnp.float32),
            pltpu.SemaphoreType.DMA((2,)),
            pltpu.SemaphoreType.DMA((2,)),
        ],
    )
    return f(table, seq_f, met_f)


def kernel(seq, met, table):
    seq_f = seq.reshape(N)
    met_f = met.reshape(N)
    flat = _run(table, seq_f, met_f)
    a = flat.reshape(L_SEQ, NGROUPS, NW, 8, 128)
    a = a.transpose(2, 4, 0, 1, 3)
    return a.reshape(B_SEQ, L_SEQ, DIM)
